# baseline (speedup 1.0000x reference)
import functools

import jax
import jax.numpy as jnp
import numpy as np
from jax import lax
from jax.experimental import pallas as pl
from jax.experimental.pallas import tpu as pltpu
from jax.experimental.pallas import tpu_sc as plsc

_N = 5000
_NP = 5120
_T = 512
_NT = _NP // _T
_SIGMA = 0.5
_D = 128

_PAIRS = np.array([(a, b) for a in range(_NT) for b in range(a, _NT)],
                  dtype=np.int32).T.copy()
_NUM_TILES = _PAIRS.shape[1]

_SC_INFO = plsc.get_sparse_core_info()
_NW = _SC_INFO.num_cores * _SC_INFO.num_subcores
_BPW = _NP // _NW
_SC_MESH = plsc.VectorSubcoreMesh(core_axis_name="c", subcore_axis_name="s")


@functools.partial(
    pl.kernel, mesh=_SC_MESH,
    out_type=jax.ShapeDtypeStruct((_NP, _D), jnp.float32),
    scratch_types=[
        pltpu.VMEM((_BPW,), jnp.int32),
        pltpu.VMEM((_BPW, _D), jnp.float32),
        pltpu.SemaphoreType.DMA,
    ],
)
def _sc_gather(table_hbm, idx_hbm, out_hbm, idx_v, rows_v, sem):
    wid = lax.axis_index("s") * _SC_INFO.num_cores + lax.axis_index("c")
    base = wid * _BPW
    pltpu.sync_copy(idx_hbm.at[pl.ds(base, _BPW)], idx_v)
    pltpu.async_copy(table_hbm.at[idx_v], rows_v, sem).wait()
    pltpu.sync_copy(rows_v, out_hbm.at[pl.ds(base, _BPW)])


@functools.partial(
    pl.kernel, mesh=_SC_MESH,
    out_type=jax.ShapeDtypeStruct((_NP, _D), jnp.float32),
    scratch_types=[
        pltpu.VMEM((_BPW,), jnp.int32),
        pltpu.VMEM((_BPW, _D), jnp.float32),
        pltpu.SemaphoreType.DMA,
    ],
)
def _sc_scatter(vals_hbm, idx_hbm, out_hbm, idx_v, rows_v, sem):
    wid = lax.axis_index("s") * _SC_INFO.num_cores + lax.axis_index("c")
    base = wid * _BPW
    pltpu.sync_copy(idx_hbm.at[pl.ds(base, _BPW)], idx_v)
    pltpu.sync_copy(vals_hbm.at[pl.ds(base, _BPW)], rows_v)
    pltpu.async_copy(rows_v, out_hbm.at[idx_v], sem).wait()


_NB = 128
_CH = 8


def _rank_kernel(cx_ref, dest_ref, rank_vmem, bin_vmem):
    lanes = lax.broadcasted_iota(jnp.int32, (1, _NB), 1).astype(jnp.float32)

    def pass1(k, run):
        cx8 = cx_ref[pl.ds(k * _CH, _CH), :]
        bin8 = jnp.clip(jnp.floor(cx8 * _NB), 0.0, _NB - 1.0)
        onehot = (bin8 == lanes).astype(jnp.float32)
        s = onehot
        for d in (1, 2, 4):
            s = s + jnp.concatenate(
                [jnp.zeros((d, _NB), jnp.float32), s[:-d, :]], axis=0)
        csum = s - onehot
        within = jnp.sum(onehot * csum, axis=1, keepdims=True)
        carried = jnp.sum(onehot * jnp.broadcast_to(run, (_CH, _NB)),
                          axis=1, keepdims=True)
        rank_vmem[pl.ds(k * _CH, _CH), :] = within + carried
        bin_vmem[pl.ds(k * _CH, _CH), :] = bin8
        return run + jnp.sum(onehot, axis=0, keepdims=True)

    counts = lax.fori_loop(0, _NP // _CH, pass1,
                           jnp.zeros((1, _NB), jnp.float32))

    s = counts
    for d in (1, 2, 4, 8, 16, 32, 64):
        s = s + jnp.concatenate(
            [jnp.zeros((1, d), jnp.float32), s[:, :-d]], axis=1)
    base = s - counts

    def pass2(k, _):
        bin8 = bin_vmem[pl.ds(k * _CH, _CH), :]
        onehot = (bin8 == lanes).astype(jnp.float32)
        base8 = jnp.sum(onehot * jnp.broadcast_to(base, (_CH, _NB)),
                        axis=1, keepdims=True)
        dest_ref[pl.ds(k * _CH, _CH), :] = \
            (base8 + rank_vmem[pl.ds(k * _CH, _CH), :]).astype(jnp.int32)
        return 0

    lax.fori_loop(0, _NP // _CH, pass2, 0)


def _decode_rows(rows):
    cx = rows[:, 0:1] * 100.0
    cy = rows[:, 1:2] * 100.0
    w = rows[:, 2:3] * 10.0 + 1e-3
    h = rows[:, 3:4] * 10.0 + 1e-3
    return (cx - w * 0.5, cx + w * 0.5, cy - h * 0.5, cy + h * 0.5,
            w * h, rows[:, 4:5])


def _decode_cols(cols):
    cx = cols[0:1, :] * 100.0
    cy = cols[1:2, :] * 100.0
    w = cols[2:3, :] * 10.0 + 1e-3
    h = cols[3:4, :] * 10.0 + 1e-3
    return (cx - w * 0.5, cx + w * 0.5, cy - h * 0.5, cy + h * 0.5,
            w * h + 1e-8, cols[4:5, :])


def _tile_kernel(tab_ref, rows_ref, cols_ref, accr_ref, accc_ref,
                 maxx2_ref, minx1_ref):
    t = pl.program_id(0)
    a = tab_ref[0, t]
    b = tab_ref[1, t]

    @pl.when(t == 0)
    def _init():
        accr_ref[...] = jnp.zeros((_NP, 128), jnp.float32)
        accc_ref[...] = jnp.zeros((8, _NP), jnp.float32)
        allr = rows_ref[...]
        cx = allr[:, 0:1] * 100.0
        wd = allr[:, 2:3] * 10.0 + 1e-3
        x1 = cx - wd * 0.5
        x2 = cx + wd * 0.5
        for blk in range(_NT):
            maxx2_ref[blk] = jnp.max(x2[blk * _T:(blk + 1) * _T, 0])
            minx1_ref[blk] = jnp.min(x1[blk * _T:(blk + 1) * _T, 0])

    live = jnp.logical_or(a == b, minx1_ref[b] <= maxx2_ref[a])

    @pl.when(live)
    def _compute():
        rows = rows_ref[pl.ds(a * _T, _T), :]
        cols = cols_ref[:, pl.ds(b * _T, _T)]
        x1r, x2r, y1r, y2r, ar, sr = _decode_rows(rows)
        x1c, x2c, y1c, y2c, ac, sc = _decode_cols(cols)

        iw = jnp.maximum(jnp.minimum(x2r, x2c) - jnp.maximum(x1r, x1c), 0.0)
        ih = jnp.maximum(jnp.minimum(y2r, y2c) - jnp.maximum(y1r, y1c), 0.0)
        inter = iw * ih
        iou = inter / ((ar + ac) - inter)
        iou2 = iou * iou

        rsel = jnp.where(sc > sr, iou2, 0.0)
        w = _T
        while w > 128:
            w //= 2
            rsel = rsel[:, :w] + rsel[:, w:]
        accr_ref[pl.ds(a * _T, _T), :] += rsel

        @pl.when(b != a)
        def _cols():
            csel = jnp.where(sr > sc, iou2, 0.0)
            hgt = _T
            while hgt > 8:
                hgt //= 2
                csel = csel[:hgt, :] + csel[hgt:, :]
            accc_ref[:, pl.ds(b * _T, _T)] += csel


def _final_kernel(accr_ref, accct_ref, s_ref, out_ref):
    total = (jnp.sum(accr_ref[...], axis=1, keepdims=True)
             + jnp.sum(accct_ref[...], axis=1, keepdims=True))
    new = s_ref[...] * jnp.exp(-total / _SIGMA)
    out_ref[...] = jnp.broadcast_to(jnp.where(new > 0.1, new, 0.0), (_NP, _D))


@jax.jit
def kernel(boxes, scores):
    pw = (0, _NP - _N)
    feats0 = jnp.pad(jnp.stack(
        [jnp.pad(boxes[:, 0], pw, constant_values=1e4),
         jnp.pad(boxes[:, 1], pw), jnp.pad(boxes[:, 2], pw),
         jnp.pad(boxes[:, 3], pw),
         jnp.pad(scores, pw, constant_values=-1.0)],
        axis=1), ((0, 0), (0, _D - 5)))

    dest = pl.pallas_call(
        _rank_kernel,
        out_shape=jax.ShapeDtypeStruct((_NP, 1), jnp.int32),
        scratch_shapes=[
            pltpu.VMEM((_NP, 1), jnp.float32),
            pltpu.VMEM((_NP, 1), jnp.float32),
        ],
    )(feats0[:, 0:1]).reshape(_NP)

    feats = _sc_scatter(feats0, dest)
    spad = feats[:, 4]
    cols = feats[:, :8].T

    accr, accc = pl.pallas_call(
        _tile_kernel,
        grid_spec=pltpu.PrefetchScalarGridSpec(
            num_scalar_prefetch=1,
            grid=(_NUM_TILES,),
            in_specs=[
                pl.BlockSpec((_NP, _D), lambda t, tab: (0, 0)),
                pl.BlockSpec((8, _NP), lambda t, tab: (0, 0)),
            ],
            out_specs=[
                pl.BlockSpec((_NP, 128), lambda t, tab: (0, 0)),
                pl.BlockSpec((8, _NP), lambda t, tab: (0, 0)),
            ],
            scratch_shapes=[
                pltpu.SMEM((_NT,), jnp.float32),
                pltpu.SMEM((_NT,), jnp.float32),
            ],
        ),
        out_shape=[
            jax.ShapeDtypeStruct((_NP, 128), jnp.float32),
            jax.ShapeDtypeStruct((8, _NP), jnp.float32),
        ],
    )(jnp.asarray(_PAIRS), feats, cols)

    out_sorted = pl.pallas_call(
        _final_kernel,
        out_shape=jax.ShapeDtypeStruct((_NP, _D), jnp.float32),
    )(accr, accc.T, spad.reshape(_NP, 1))

    out = _sc_gather(out_sorted, dest)
    return out[:_N, 0]


# speedup vs baseline: 3.1749x; 3.1749x over previous
import functools

import jax
import jax.numpy as jnp
import numpy as np
from jax import lax
from jax.experimental import pallas as pl
from jax.experimental.pallas import tpu as pltpu
from jax.experimental.pallas import tpu_sc as plsc

_N = 5000
_NP = 5120
_T = 512
_NT = _NP // _T
_SIGMA = 0.5
_D = 128

_PAIRS = np.array([(a, b) for a in range(_NT) for b in range(a, _NT)],
                  dtype=np.int32).T.copy()
_NUM_TILES = _PAIRS.shape[1]

_SC_INFO = plsc.get_sparse_core_info()
_NW = _SC_INFO.num_cores * _SC_INFO.num_subcores
_BPW = _NP // _NW
_SC_MESH = plsc.VectorSubcoreMesh(core_axis_name="c", subcore_axis_name="s")


@functools.partial(
    pl.kernel, mesh=_SC_MESH,
    out_type=jax.ShapeDtypeStruct((_NP, _D), jnp.float32),
    scratch_types=[
        pltpu.VMEM((_BPW,), jnp.int32),
        pltpu.VMEM((_BPW, _D), jnp.float32),
        pltpu.SemaphoreType.DMA,
    ],
)
def _sc_gather(table_hbm, idx_hbm, out_hbm, idx_v, rows_v, sem):
    wid = lax.axis_index("s") * _SC_INFO.num_cores + lax.axis_index("c")
    base = wid * _BPW
    pltpu.sync_copy(idx_hbm.at[pl.ds(base, _BPW)], idx_v)
    pltpu.async_copy(table_hbm.at[idx_v], rows_v, sem).wait()
    pltpu.sync_copy(rows_v, out_hbm.at[pl.ds(base, _BPW)])


@functools.partial(
    pl.kernel, mesh=_SC_MESH,
    out_type=jax.ShapeDtypeStruct((_NP, _D), jnp.float32),
    scratch_types=[
        pltpu.VMEM((_BPW,), jnp.int32),
        pltpu.VMEM((_BPW, _D), jnp.float32),
        pltpu.SemaphoreType.DMA,
    ],
)
def _sc_scatter(vals_hbm, idx_hbm, out_hbm, idx_v, rows_v, sem):
    wid = lax.axis_index("s") * _SC_INFO.num_cores + lax.axis_index("c")
    base = wid * _BPW
    pltpu.sync_copy(idx_hbm.at[pl.ds(base, _BPW)], idx_v)
    pltpu.sync_copy(vals_hbm.at[pl.ds(base, _BPW)], rows_v)
    pltpu.async_copy(rows_v, out_hbm.at[idx_v], sem).wait()


_NB = 128
_CH = 8


def _rank_kernel(cx_ref, dest_ref):
    lanes = lax.broadcasted_iota(jnp.int32, (1, _NB), 1).astype(jnp.float32)
    cx = cx_ref[...]
    bins = jnp.clip(jnp.floor(cx * _NB), 0.0, _NB - 1.0)
    oh = (bins == lanes).astype(jnp.float32)

    s = oh
    d = 1
    while d < _NP:
        s = s + jnp.concatenate(
            [jnp.zeros((d, _NB), jnp.float32), s[:-d, :]], axis=0)
        d *= 2
    rank = jnp.sum(oh * (s - oh), axis=1, keepdims=True)

    counts = s[_NP - 1:_NP, :]
    b = counts
    d = 1
    while d < _NB:
        b = b + jnp.concatenate(
            [jnp.zeros((1, d), jnp.float32), b[:, :-d]], axis=1)
        d *= 2
    base = b - counts

    dest = rank + jnp.sum(oh * jnp.broadcast_to(base, (_NP, _NB)),
                          axis=1, keepdims=True)
    dest_ref[...] = dest.astype(jnp.int32)


def _decode_rows(rows):
    cx = rows[:, 0:1] * 100.0
    cy = rows[:, 1:2] * 100.0
    w = rows[:, 2:3] * 10.0 + 1e-3
    h = rows[:, 3:4] * 10.0 + 1e-3
    return (cx - w * 0.5, cx + w * 0.5, cy - h * 0.5, cy + h * 0.5,
            w * h, rows[:, 4:5])


def _decode_cols(cols):
    cx = cols[0:1, :] * 100.0
    cy = cols[1:2, :] * 100.0
    w = cols[2:3, :] * 10.0 + 1e-3
    h = cols[3:4, :] * 10.0 + 1e-3
    return (cx - w * 0.5, cx + w * 0.5, cy - h * 0.5, cy + h * 0.5,
            w * h + 1e-8, cols[4:5, :])


def _tile_kernel(tab_ref, rows_ref, cols_ref, accr_ref, accc_ref,
                 maxx2_ref, minx1_ref):
    t = pl.program_id(0)
    a = tab_ref[0, t]
    b = tab_ref[1, t]

    @pl.when(t == 0)
    def _init():
        accr_ref[...] = jnp.zeros((_NP, 128), jnp.float32)
        accc_ref[...] = jnp.zeros((8, _NP), jnp.float32)
        allr = rows_ref[...]
        cx = allr[:, 0:1] * 100.0
        wd = allr[:, 2:3] * 10.0 + 1e-3
        x1 = cx - wd * 0.5
        x2 = cx + wd * 0.5
        for blk in range(_NT):
            maxx2_ref[blk] = jnp.max(x2[blk * _T:(blk + 1) * _T, 0])
            minx1_ref[blk] = jnp.min(x1[blk * _T:(blk + 1) * _T, 0])

    live = jnp.logical_or(a == b, minx1_ref[b] <= maxx2_ref[a])

    @pl.when(live)
    def _compute():
        rows = rows_ref[pl.ds(a * _T, _T), :]
        cols = cols_ref[:, pl.ds(b * _T, _T)]
        x1r, x2r, y1r, y2r, ar, sr = _decode_rows(rows)
        x1c, x2c, y1c, y2c, ac, sc = _decode_cols(cols)

        iw = jnp.maximum(jnp.minimum(x2r, x2c) - jnp.maximum(x1r, x1c), 0.0)
        ih = jnp.maximum(jnp.minimum(y2r, y2c) - jnp.maximum(y1r, y1c), 0.0)
        inter = iw * ih
        iou = inter / ((ar + ac) - inter)
        iou2 = iou * iou

        rsel = jnp.where(sc > sr, iou2, 0.0)
        w = _T
        while w > 128:
            w //= 2
            rsel = rsel[:, :w] + rsel[:, w:]
        accr_ref[pl.ds(a * _T, _T), :] += rsel

        @pl.when(b != a)
        def _cols():
            csel = jnp.where(sr > sc, iou2, 0.0)
            hgt = _T
            while hgt > 8:
                hgt //= 2
                csel = csel[:hgt, :] + csel[hgt:, :]
            accc_ref[:, pl.ds(b * _T, _T)] += csel


def _final_kernel(accr_ref, accct_ref, s_ref, out_ref):
    total = (jnp.sum(accr_ref[...], axis=1, keepdims=True)
             + jnp.sum(accct_ref[...], axis=1, keepdims=True))
    new = s_ref[...] * jnp.exp(-total / _SIGMA)
    out_ref[...] = jnp.broadcast_to(jnp.where(new > 0.1, new, 0.0), (_NP, _D))


@jax.jit
def kernel(boxes, scores):
    pw = (0, _NP - _N)
    feats0 = jnp.pad(jnp.stack(
        [jnp.pad(boxes[:, 0], pw, constant_values=1e4),
         jnp.pad(boxes[:, 1], pw), jnp.pad(boxes[:, 2], pw),
         jnp.pad(boxes[:, 3], pw),
         jnp.pad(scores, pw, constant_values=-1.0)],
        axis=1), ((0, 0), (0, _D - 5)))

    dest = pl.pallas_call(
        _rank_kernel,
        out_shape=jax.ShapeDtypeStruct((_NP, 1), jnp.int32),
    )(feats0[:, 0:1]).reshape(_NP)

    feats = _sc_scatter(feats0, dest)
    spad = feats[:, 4]
    cols = feats[:, :8].T

    accr, accc = pl.pallas_call(
        _tile_kernel,
        grid_spec=pltpu.PrefetchScalarGridSpec(
            num_scalar_prefetch=1,
            grid=(_NUM_TILES,),
            in_specs=[
                pl.BlockSpec((_NP, _D), lambda t, tab: (0, 0)),
                pl.BlockSpec((8, _NP), lambda t, tab: (0, 0)),
            ],
            out_specs=[
                pl.BlockSpec((_NP, 128), lambda t, tab: (0, 0)),
                pl.BlockSpec((8, _NP), lambda t, tab: (0, 0)),
            ],
            scratch_shapes=[
                pltpu.SMEM((_NT,), jnp.float32),
                pltpu.SMEM((_NT,), jnp.float32),
            ],
        ),
        out_shape=[
            jax.ShapeDtypeStruct((_NP, 128), jnp.float32),
            jax.ShapeDtypeStruct((8, _NP), jnp.float32),
        ],
    )(jnp.asarray(_PAIRS), feats, cols)

    out_sorted = pl.pallas_call(
        _final_kernel,
        out_shape=jax.ShapeDtypeStruct((_NP, _D), jnp.float32),
    )(accr, accc.T, spad.reshape(_NP, 1))

    out = _sc_gather(out_sorted, dest)
    return out[:_N, 0]


# speedup vs baseline: 3.5928x; 1.1316x over previous
import functools

import jax
import jax.numpy as jnp
import numpy as np
from jax import lax
from jax.experimental import pallas as pl
from jax.experimental.pallas import tpu as pltpu

_N = 5000
_NP = 5120
_T = 512
_NT = _NP // _T
_SIGMA = 0.5

_PAIRS = np.array([(a, b) for a in range(_NT) for b in range(a, _NT)],
                  dtype=np.int32).T.copy()
_NUM_TILES = _PAIRS.shape[1]


def _decode_rows(rows):
    cx = rows[:, 0:1] * 100.0
    cy = rows[:, 1:2] * 100.0
    w = rows[:, 2:3] * 10.0 + 1e-3
    h = rows[:, 3:4] * 10.0 + 1e-3
    return (cx - w * 0.5, cx + w * 0.5, cy - h * 0.5, cy + h * 0.5,
            w * h, rows[:, 4:5])


def _decode_cols(cols):
    cx = cols[0:1, :] * 100.0
    cy = cols[1:2, :] * 100.0
    w = cols[2:3, :] * 10.0 + 1e-3
    h = cols[3:4, :] * 10.0 + 1e-3
    return (cx - w * 0.5, cx + w * 0.5, cy - h * 0.5, cy + h * 0.5,
            w * h + 1e-8, cols[4:5, :])


def _tile_kernel(tab_ref, rows_ref, cols_ref, accr_ref, accc_ref,
                 maxx2_ref, minx1_ref):
    t = pl.program_id(0)
    a = tab_ref[0, t]
    b = tab_ref[1, t]

    @pl.when(t == 0)
    def _init():
        accr_ref[...] = jnp.zeros((_NP, 128), jnp.float32)
        accc_ref[...] = jnp.zeros((8, _NP), jnp.float32)
        allr = rows_ref[...]
        cx = allr[:, 0:1] * 100.0
        wd = allr[:, 2:3] * 10.0 + 1e-3
        x1 = cx - wd * 0.5
        x2 = cx + wd * 0.5
        for blk in range(_NT):
            maxx2_ref[blk] = jnp.max(x2[blk * _T:(blk + 1) * _T, 0])
            minx1_ref[blk] = jnp.min(x1[blk * _T:(blk + 1) * _T, 0])

    live = jnp.logical_or(a == b, minx1_ref[b] <= maxx2_ref[a])

    @pl.when(live)
    def _compute():
        rows = rows_ref[pl.ds(a * _T, _T), :]
        cols = cols_ref[:, pl.ds(b * _T, _T)]
        x1r, x2r, y1r, y2r, ar, sr = _decode_rows(rows)
        x1c, x2c, y1c, y2c, ac, sc = _decode_cols(cols)

        iw = jnp.maximum(jnp.minimum(x2r, x2c) - jnp.maximum(x1r, x1c), 0.0)
        ih = jnp.maximum(jnp.minimum(y2r, y2c) - jnp.maximum(y1r, y1c), 0.0)
        inter = iw * ih
        iou = inter / ((ar + ac) - inter)
        iou2 = iou * iou

        rsel = jnp.where(sc > sr, iou2, 0.0)
        w = _T
        while w > 128:
            w //= 2
            rsel = rsel[:, :w] + rsel[:, w:]
        accr_ref[pl.ds(a * _T, _T), :] += rsel

        @pl.when(b != a)
        def _cols():
            csel = jnp.where(sr > sc, iou2, 0.0)
            hgt = _T
            while hgt > 8:
                hgt //= 2
                csel = csel[:hgt, :] + csel[hgt:, :]
            accc_ref[:, pl.ds(b * _T, _T)] += csel


def _final_kernel(accr_ref, accct_ref, s_ref, out_ref):
    total = (jnp.sum(accr_ref[...], axis=1, keepdims=True)
             + jnp.sum(accct_ref[...], axis=1, keepdims=True))
    new = s_ref[...] * jnp.exp(-total / _SIGMA)
    out_ref[...] = jnp.where(new > 0.1, new, 0.0)


@jax.jit
def kernel(boxes, scores):
    idx = lax.iota(jnp.int32, _N)
    cx, cy, wb, hb, ssc, order = lax.sort(
        (boxes[:, 0], boxes[:, 1], boxes[:, 2], boxes[:, 3], scores, idx),
        num_keys=1)

    pw = (0, _NP - _N)
    feats = jnp.stack(
        [jnp.pad(cx, pw, constant_values=1e4), jnp.pad(cy, pw),
         jnp.pad(wb, pw), jnp.pad(hb, pw),
         jnp.pad(ssc, pw, constant_values=-1.0),
         jnp.zeros(_NP, jnp.float32), jnp.zeros(_NP, jnp.float32),
         jnp.zeros(_NP, jnp.float32)], axis=1)
    spad = feats[:, 4]
    cols = feats.T

    accr, accc = pl.pallas_call(
        _tile_kernel,
        grid_spec=pltpu.PrefetchScalarGridSpec(
            num_scalar_prefetch=1,
            grid=(_NUM_TILES,),
            in_specs=[
                pl.BlockSpec((_NP, 8), lambda t, tab: (0, 0)),
                pl.BlockSpec((8, _NP), lambda t, tab: (0, 0)),
            ],
            out_specs=[
                pl.BlockSpec((_NP, 128), lambda t, tab: (0, 0)),
                pl.BlockSpec((8, _NP), lambda t, tab: (0, 0)),
            ],
            scratch_shapes=[
                pltpu.SMEM((_NT,), jnp.float32),
                pltpu.SMEM((_NT,), jnp.float32),
            ],
        ),
        out_shape=[
            jax.ShapeDtypeStruct((_NP, 128), jnp.float32),
            jax.ShapeDtypeStruct((8, _NP), jnp.float32),
        ],
    )(jnp.asarray(_PAIRS), feats, cols)

    out = pl.pallas_call(
        _final_kernel,
        out_shape=jax.ShapeDtypeStruct((_NP, 1), jnp.float32),
    )(accr, accc.T, spad.reshape(_NP, 1))
    return jnp.zeros((_N,), jnp.float32).at[order].set(out[:_N, 0])
